# baseline (speedup 1.0000x reference)
import functools

import jax
import jax.numpy as jnp
from jax import lax
from jax.experimental import pallas as pl
from jax.experimental.pallas import tpu as pltpu
from jax.experimental.pallas import tpu_sc as plsc

VOCAB = 1000000
HIDDEN = 64
ROWS = 4096
COLS = 200
NUM_WORKERS = 32
ROWS_PER_W = ROWS // NUM_WORKERS
RCHUNK = 2
N_CHUNKS = ROWS_PER_W // RCHUNK
N_PAIR = N_CHUNKS // 2
HALF = COLS // 2


@functools.partial(
    pl.kernel,
    out_type=jax.ShapeDtypeStruct((ROWS, COLS, HIDDEN), jnp.float32),
    mesh=plsc.VectorSubcoreMesh(core_axis_name="c", subcore_axis_name="s"),
    compiler_params=pltpu.CompilerParams(use_tc_tiling_on_sc=False),
    scratch_types=[
        pltpu.VMEM((ROWS_PER_W, COLS), jnp.int32),
        pltpu.VMEM((2, RCHUNK, COLS, HIDDEN), jnp.float32),
        pltpu.SemaphoreType.DMA,
        pltpu.SemaphoreType.DMA,
    ],
)
def _emb_lookup(idx_hbm, table_hbm, out_hbm, idx_v, rows, sem_g, sem_o):
    wid = lax.axis_index("s") * 2 + lax.axis_index("c")
    base_row = wid * ROWS_PER_W
    pltpu.sync_copy(idx_hbm.at[pl.ds(base_row, ROWS_PER_W)], idx_v)

    def pair(i, carry):
        for b in range(2):
            c = i * 2 + b
            r0 = c * RCHUNK

            @pl.when(i > 0)
            def _():
                pltpu.make_async_copy(
                    rows.at[b], out_hbm.at[pl.ds(base_row, RCHUNK)], sem_o
                ).wait()

            gathers = [
                pltpu.async_copy(
                    table_hbm.at[idx_v.at[r0 + rr]],
                    rows.at[b, rr],
                    sem_g,
                )
                for rr in range(RCHUNK)
            ]
            for g in gathers:
                g.wait()
            pltpu.async_copy(
                rows.at[b], out_hbm.at[pl.ds(base_row + r0, RCHUNK)], sem_o
            )
        return carry

    lax.fori_loop(0, N_PAIR, pair, 0)
    for b in range(2):
        pltpu.make_async_copy(
            rows.at[b], out_hbm.at[pl.ds(base_row, RCHUNK)], sem_o
        ).wait()


def kernel(input_ids, emb_weight):
    if input_ids.dtype != jnp.int32:
        input_ids = input_ids.astype(jnp.int32)
    return _emb_lookup(input_ids, emb_weight)


# speedup vs baseline: 1.0006x; 1.0006x over previous
import functools

import jax
import jax.numpy as jnp
from jax import lax
from jax.experimental import pallas as pl
from jax.experimental.pallas import tpu as pltpu
from jax.experimental.pallas import tpu_sc as plsc

VOCAB = 1000000
HIDDEN = 64
ROWS = 4096
COLS = 200
NUM_WORKERS = 32
ROWS_PER_W = ROWS // NUM_WORKERS
RCHUNK = 2
N_CHUNKS = ROWS_PER_W // RCHUNK
N_PAIR = N_CHUNKS // 2


@functools.partial(
    pl.kernel,
    out_type=jax.ShapeDtypeStruct((ROWS, COLS, HIDDEN), jnp.float32),
    mesh=plsc.VectorSubcoreMesh(core_axis_name="c", subcore_axis_name="s"),
    compiler_params=pltpu.CompilerParams(use_tc_tiling_on_sc=False),
    scratch_types=[
        pltpu.VMEM((ROWS_PER_W, COLS), jnp.int32),
        pltpu.VMEM((2, RCHUNK, COLS, HIDDEN), jnp.float32),
        pltpu.SemaphoreType.DMA,
        pltpu.SemaphoreType.DMA,
    ],
)
def _emb_lookup(idx_hbm, table_hbm, out_hbm, idx_v, rows, sem_g, sem_o):
    wid = lax.axis_index("s") * 2 + lax.axis_index("c")
    base_row = wid * ROWS_PER_W
    pltpu.sync_copy(idx_hbm.at[pl.ds(base_row, ROWS_PER_W)], idx_v)

    def pair(i, carry):
        for b in range(2):
            c = i * 2 + b
            r0 = c * RCHUNK

            @pl.when(i > 0)
            def _():
                pltpu.make_async_copy(
                    rows.at[b], out_hbm.at[pl.ds(base_row, RCHUNK)], sem_o
                ).wait()

            gathers = [
                pltpu.async_copy(
                    table_hbm.at[idx_v.at[r0 + rr]],
                    rows.at[b, rr],
                    sem_g,
                )
                for rr in range(RCHUNK)
            ]
            for g in gathers:
                g.wait()
            pltpu.async_copy(
                rows.at[b], out_hbm.at[pl.ds(base_row + r0, RCHUNK)], sem_o
            )
        return carry

    lax.fori_loop(0, N_PAIR, pair, 0)
    for b in range(2):
        pltpu.make_async_copy(
            rows.at[b], out_hbm.at[pl.ds(base_row, RCHUNK)], sem_o
        ).wait()


def kernel(input_ids, emb_weight):
    if input_ids.dtype != jnp.int32:
        input_ids = input_ids.astype(jnp.int32)
    return _emb_lookup(input_ids, emb_weight)
